# baseline (speedup 1.0000x reference)
import jax
import jax.numpy as jnp
from jax.experimental import pallas as pl


def _add_kernel(x_ref, pe_ref, out_ref):
    out_ref[...] = x_ref[...] + pe_ref[...]


def kernel(x, pe):
    batch, seq_len, d_model = x.shape
    s_blk = 2048
    grid = (seq_len // s_blk, batch)
    return pl.pallas_call(
        _add_kernel,
        grid=grid,
        in_specs=[
            pl.BlockSpec((1, s_blk, d_model), lambda i, b: (b, i, 0)),
            pl.BlockSpec((s_blk, d_model), lambda i, b: (i, 0)),
        ],
        out_specs=pl.BlockSpec((1, s_blk, d_model), lambda i, b: (b, i, 0)),
        out_shape=jax.ShapeDtypeStruct(x.shape, x.dtype),
    )(x, pe[:seq_len])
